# baseline (speedup 1.0000x reference)
import jax
import jax.numpy as jnp
from jax import lax
from jax.experimental import pallas as pl
from jax.experimental.pallas import tpu as pltpu
from jax.experimental.pallas import tpu_sc as plsc

N_WORKERS = 32
P_PER_W = 2
EDGE_CHUNK = 6400
UNROLL = 16
LANES = 16


def _matmul_pack_body(w_ref, x_ref, o_ref):
    psT = lax.dot_general(
        w_ref[...], x_ref[...],
        dimension_numbers=(((0,), (1,)), ((), ())),
        preferred_element_type=jnp.float32)
    bits = lax.bitcast_convert_type(psT, jnp.int32)
    rnd = bits + 0x7FFF + lax.bitwise_and(
        lax.shift_right_logical(bits, 16), 1)
    half = rnd.shape[0] // 2
    lo = lax.shift_right_logical(rnd[:half], 16)
    hi = lax.bitwise_and(rnd[half:], -65536)
    o_ref[...] = lax.bitwise_or(hi, lo)


def _transpose_body(i_ref, o_ref):
    o_ref[...] = i_ref[...].T


def _make_sc_edge_kernel(n, d, e):
    n_chunks = e // EDGE_CHUNK
    n_groups = EDGE_CHUNK // LANES
    def process_chunk(ebuf, cpair, outc):
        mask_hi = jnp.full((LANES,), -65536, jnp.int32)
        @plsc.parallel_loop(0, n_groups, unroll=UNROLL)
        def group_step(g):
            sl = pl.ds(g * LANES, LANES)
            r16 = ebuf[0, sl]
            c16 = ebuf[1, sl]
            v16 = plsc.bitcast(ebuf[2, sl], jnp.float32)
            for p in range(P_PER_W):
                pv = jnp.full((LANES,), p, jnp.int32)
                g16 = plsc.load_gather(cpair, [pv, c16])
                f_lo = plsc.bitcast(lax.shift_left(g16, 16), jnp.float32)
                f_hi = plsc.bitcast(lax.bitwise_and(g16, mask_hi),
                                    jnp.float32)
                jlo = jnp.full((LANES,), p, jnp.int32)
                jhi = jnp.full((LANES,), p + P_PER_W, jnp.int32)
                plsc.addupdate_scatter(outc, [jlo, r16], f_lo * v16)
                plsc.addupdate_scatter(outc, [jhi, r16], f_hi * v16)

    def body(ps_hbm, edges_hbm, out_hbm, cpair, outc, ebuf0, ebuf1,
             sem0, sem1):
        w = lax.axis_index("c") * 16 + lax.axis_index("s")
        p0 = w * P_PER_W

        pltpu.sync_copy(ps_hbm.at[pl.ds(p0, P_PER_W)], cpair)

        zero = jnp.zeros((LANES,), jnp.float32)

        def zero_step(i, carry):
            for u in range(5):
                for j in range(2 * P_PER_W):
                    outc[j, pl.ds((i * 5 + u) * LANES, LANES)] = zero
            return carry

        lax.fori_loop(0, n // (5 * LANES), zero_step, 0)

        def chunk_src(k):
            return edges_hbm.at[:, pl.ds(k * EDGE_CHUNK, EDGE_CHUNK)]

        pltpu.async_copy(chunk_src(0), ebuf0, sem0)

        def outer(k2, carry):
            k = k2 * 2
            pltpu.make_async_copy(chunk_src(0), ebuf0, sem0).wait()
            pltpu.async_copy(chunk_src(k + 1), ebuf1, sem1)
            process_chunk(ebuf0, cpair, outc)
            pltpu.make_async_copy(chunk_src(0), ebuf1, sem1).wait()

            @pl.when(k + 2 < n_chunks)
            def _():
                pltpu.async_copy(chunk_src(k + 2), ebuf0, sem0)

            process_chunk(ebuf1, cpair, outc)
            return carry

        lax.fori_loop(0, n_chunks // 2, outer, 0)

        pltpu.sync_copy(outc.at[pl.ds(0, P_PER_W)],
                        out_hbm.at[pl.ds(p0, P_PER_W)])
        pltpu.sync_copy(outc.at[pl.ds(P_PER_W, P_PER_W)],
                        out_hbm.at[pl.ds(d // 2 + p0, P_PER_W)])

    mesh = plsc.VectorSubcoreMesh(core_axis_name="c", subcore_axis_name="s")
    return pl.kernel(
        body,
        out_type=jax.ShapeDtypeStruct((d, n), jnp.float32),
        mesh=mesh,
        compiler_params=pltpu.CompilerParams(needs_layout_passes=False),
        scratch_types=[
            pltpu.VMEM((P_PER_W, n), jnp.int32),
            pltpu.VMEM((2 * P_PER_W, n), jnp.float32),
            pltpu.VMEM((3, EDGE_CHUNK), jnp.int32),
            pltpu.VMEM((3, EDGE_CHUNK), jnp.int32),
            pltpu.SemaphoreType.DMA,
            pltpu.SemaphoreType.DMA,
        ],
    )


def kernel(x, W0, adj_row, adj_col, adj_val):
    n, _ = x.shape
    d = W0.shape[1]
    e = adj_row.shape[0]

    edges = jnp.stack(
        [adj_row, adj_col, lax.bitcast_convert_type(adj_val, jnp.int32)])

    ps_packed = pl.pallas_call(
        _matmul_pack_body,
        out_shape=jax.ShapeDtypeStruct((d // 2, n), jnp.int32),
    )(W0, x)

    outT = _make_sc_edge_kernel(n, d, e)(ps_packed, edges)

    out = pl.pallas_call(
        _transpose_body,
        out_shape=jax.ShapeDtypeStruct((n, d), jnp.float32),
    )(outT)
    return out


# speedup vs baseline: 1.0040x; 1.0040x over previous
import jax
import jax.numpy as jnp
from jax import lax
from jax.experimental import pallas as pl
from jax.experimental.pallas import tpu as pltpu
from jax.experimental.pallas import tpu_sc as plsc

N_WORKERS = 32
P_PER_W = 2
EDGE_CHUNK = 6400
UNROLL = 16
LANES = 16


def _matmul_pack_body(w_ref, x_ref, o_ref):
    psT = lax.dot_general(
        w_ref[...], x_ref[...],
        dimension_numbers=(((0,), (1,)), ((), ())),
        preferred_element_type=jnp.float32)
    bits = lax.bitcast_convert_type(psT, jnp.int32)
    rnd = bits + 0x7FFF + lax.bitwise_and(
        lax.shift_right_logical(bits, 16), 1)
    half = rnd.shape[0] // 2
    lo = lax.shift_right_logical(rnd[:half], 16)
    hi = lax.bitwise_and(rnd[half:], -65536)
    o_ref[...] = lax.bitwise_or(hi, lo)


def _transpose_body(i_ref, o_ref):
    o_ref[...] = i_ref[...].T


def _make_sc_edge_kernel(n, d, e):
    n_chunks = e // EDGE_CHUNK
    n_groups = EDGE_CHUNK // LANES
    def process_chunk(ebuf, cpair, outc):
        mask_hi = jnp.full((LANES,), -65536, jnp.int32)
        @plsc.parallel_loop(0, n_groups, unroll=UNROLL)
        def group_step(g):
            sl = pl.ds(g * LANES, LANES)
            r16 = ebuf[0, sl]
            c16 = ebuf[1, sl]
            v16 = plsc.bitcast(ebuf[2, sl], jnp.float32)
            for p in range(P_PER_W):
                pv = jnp.full((LANES,), p, jnp.int32)
                g16 = plsc.load_gather(cpair, [pv, c16])
                f_lo = plsc.bitcast(lax.shift_left(g16, 16), jnp.float32)
                f_hi = plsc.bitcast(lax.bitwise_and(g16, mask_hi),
                                    jnp.float32)
                jlo = jnp.full((LANES,), p, jnp.int32)
                jhi = jnp.full((LANES,), p + P_PER_W, jnp.int32)
                plsc.addupdate_scatter(outc, [jlo, r16], f_lo * v16)
                plsc.addupdate_scatter(outc, [jhi, r16], f_hi * v16)

    def body(ps_hbm, edges_hbm, out_hbm, cpair, outc, ebuf0, ebuf1,
             sem0, sem1, semc):
        w = lax.axis_index("c") * 16 + lax.axis_index("s")
        p0 = w * P_PER_W

        def chunk_src(k):
            return edges_hbm.at[:, pl.ds(k * EDGE_CHUNK, EDGE_CHUNK)]

        pltpu.async_copy(chunk_src(0), ebuf0, sem0)
        pltpu.async_copy(ps_hbm.at[pl.ds(p0, P_PER_W)], cpair, semc)

        zero = jnp.zeros((LANES,), jnp.float32)

        def zero_step(i, carry):
            for u in range(5):
                for j in range(2 * P_PER_W):
                    outc[j, pl.ds((i * 5 + u) * LANES, LANES)] = zero
            return carry

        lax.fori_loop(0, n // (5 * LANES), zero_step, 0)

        pltpu.make_async_copy(ps_hbm.at[pl.ds(p0, P_PER_W)], cpair,
                              semc).wait()

        def outer(k2, carry):
            k = k2 * 2
            pltpu.make_async_copy(chunk_src(0), ebuf0, sem0).wait()
            pltpu.async_copy(chunk_src(k + 1), ebuf1, sem1)
            process_chunk(ebuf0, cpair, outc)
            pltpu.make_async_copy(chunk_src(0), ebuf1, sem1).wait()

            @pl.when(k + 2 < n_chunks)
            def _():
                pltpu.async_copy(chunk_src(k + 2), ebuf0, sem0)

            process_chunk(ebuf1, cpair, outc)
            return carry

        lax.fori_loop(0, n_chunks // 2, outer, 0)

        pltpu.sync_copy(outc.at[pl.ds(0, P_PER_W)],
                        out_hbm.at[pl.ds(p0, P_PER_W)])
        pltpu.sync_copy(outc.at[pl.ds(P_PER_W, P_PER_W)],
                        out_hbm.at[pl.ds(d // 2 + p0, P_PER_W)])

    mesh = plsc.VectorSubcoreMesh(core_axis_name="c", subcore_axis_name="s")
    return pl.kernel(
        body,
        out_type=jax.ShapeDtypeStruct((d, n), jnp.float32),
        mesh=mesh,
        compiler_params=pltpu.CompilerParams(needs_layout_passes=False),
        scratch_types=[
            pltpu.VMEM((P_PER_W, n), jnp.int32),
            pltpu.VMEM((2 * P_PER_W, n), jnp.float32),
            pltpu.VMEM((3, EDGE_CHUNK), jnp.int32),
            pltpu.VMEM((3, EDGE_CHUNK), jnp.int32),
            pltpu.SemaphoreType.DMA,
            pltpu.SemaphoreType.DMA,
            pltpu.SemaphoreType.DMA,
        ],
    )


def kernel(x, W0, adj_row, adj_col, adj_val):
    n, _ = x.shape
    d = W0.shape[1]
    e = adj_row.shape[0]

    edges = jnp.stack(
        [adj_row, adj_col, lax.bitcast_convert_type(adj_val, jnp.int32)])

    ps_packed = pl.pallas_call(
        _matmul_pack_body,
        out_shape=jax.ShapeDtypeStruct((d // 2, n), jnp.int32),
    )(W0, x)

    outT = _make_sc_edge_kernel(n, d, e)(ps_packed, edges)

    out = pl.pallas_call(
        _transpose_body,
        out_shape=jax.ShapeDtypeStruct((n, d), jnp.float32),
    )(outT)
    return out


# speedup vs baseline: 1.1011x; 1.0967x over previous
import jax
import jax.numpy as jnp
from jax import lax
from jax.experimental import pallas as pl
from jax.experimental.pallas import tpu as pltpu
from jax.experimental.pallas import tpu_sc as plsc

N_WORKERS = 32
P_PER_W = 2
EDGE_CHUNK = 6400
UNROLL = 16
LANES = 16


def _matmul_pack_body(w_ref, x_ref, r_ref, c_ref, v_ref, o_ref, e_ref):
    e_ref[0, :] = r_ref[...]
    e_ref[1, :] = c_ref[...]
    e_ref[2, :] = lax.bitcast_convert_type(v_ref[...], jnp.int32)
    psT = lax.dot_general(
        w_ref[...], x_ref[...],
        dimension_numbers=(((0,), (1,)), ((), ())),
        preferred_element_type=jnp.float32)
    bits = lax.bitcast_convert_type(psT, jnp.int32)
    rnd = bits + 0x7FFF + lax.bitwise_and(
        lax.shift_right_logical(bits, 16), 1)
    half = rnd.shape[0] // 2
    lo = lax.shift_right_logical(rnd[:half], 16)
    hi = lax.bitwise_and(rnd[half:], -65536)
    o_ref[...] = lax.bitwise_or(hi, lo)


def _transpose_body(i_ref, o_ref):
    o_ref[...] = i_ref[...].T


def _make_sc_edge_kernel(n, d, e):
    n_chunks = e // EDGE_CHUNK
    n_groups = EDGE_CHUNK // LANES
    def process_chunk(ebuf, cpair, outc):
        mask_hi = jnp.full((LANES,), -65536, jnp.int32)
        @plsc.parallel_loop(0, n_groups, unroll=UNROLL)
        def group_step(g):
            sl = pl.ds(g * LANES, LANES)
            r16 = ebuf[0, sl]
            c16 = ebuf[1, sl]
            v16 = plsc.bitcast(ebuf[2, sl], jnp.float32)
            for p in range(P_PER_W):
                pv = jnp.full((LANES,), p, jnp.int32)
                g16 = plsc.load_gather(cpair, [pv, c16])
                f_lo = plsc.bitcast(lax.shift_left(g16, 16), jnp.float32)
                f_hi = plsc.bitcast(lax.bitwise_and(g16, mask_hi),
                                    jnp.float32)
                jlo = jnp.full((LANES,), p, jnp.int32)
                jhi = jnp.full((LANES,), p + P_PER_W, jnp.int32)
                plsc.addupdate_scatter(outc, [jlo, r16], f_lo * v16)
                plsc.addupdate_scatter(outc, [jhi, r16], f_hi * v16)

    def body(ps_hbm, edges_hbm, out_hbm, cpair, outc, ebuf0, ebuf1,
             sem0, sem1, semc):
        w = lax.axis_index("c") * 16 + lax.axis_index("s")
        p0 = w * P_PER_W

        def chunk_src(k):
            return edges_hbm.at[:, pl.ds(k * EDGE_CHUNK, EDGE_CHUNK)]

        pltpu.async_copy(chunk_src(0), ebuf0, sem0)
        pltpu.async_copy(ps_hbm.at[pl.ds(p0, P_PER_W)], cpair, semc)

        zero = jnp.zeros((LANES,), jnp.float32)

        def zero_step(i, carry):
            for u in range(5):
                for j in range(2 * P_PER_W):
                    outc[j, pl.ds((i * 5 + u) * LANES, LANES)] = zero
            return carry

        lax.fori_loop(0, n // (5 * LANES), zero_step, 0)

        pltpu.make_async_copy(ps_hbm.at[pl.ds(p0, P_PER_W)], cpair,
                              semc).wait()

        def outer(k2, carry):
            k = k2 * 2
            pltpu.make_async_copy(chunk_src(0), ebuf0, sem0).wait()
            pltpu.async_copy(chunk_src(k + 1), ebuf1, sem1)
            process_chunk(ebuf0, cpair, outc)
            pltpu.make_async_copy(chunk_src(0), ebuf1, sem1).wait()

            @pl.when(k + 2 < n_chunks)
            def _():
                pltpu.async_copy(chunk_src(k + 2), ebuf0, sem0)

            process_chunk(ebuf1, cpair, outc)
            return carry

        lax.fori_loop(0, n_chunks // 2, outer, 0)

        pltpu.sync_copy(outc.at[pl.ds(0, P_PER_W)],
                        out_hbm.at[pl.ds(p0, P_PER_W)])
        pltpu.sync_copy(outc.at[pl.ds(P_PER_W, P_PER_W)],
                        out_hbm.at[pl.ds(d // 2 + p0, P_PER_W)])

    mesh = plsc.VectorSubcoreMesh(core_axis_name="c", subcore_axis_name="s")
    return pl.kernel(
        body,
        out_type=jax.ShapeDtypeStruct((d, n), jnp.float32),
        mesh=mesh,
        compiler_params=pltpu.CompilerParams(needs_layout_passes=False),
        scratch_types=[
            pltpu.VMEM((P_PER_W, n), jnp.int32),
            pltpu.VMEM((2 * P_PER_W, n), jnp.float32),
            pltpu.VMEM((3, EDGE_CHUNK), jnp.int32),
            pltpu.VMEM((3, EDGE_CHUNK), jnp.int32),
            pltpu.SemaphoreType.DMA,
            pltpu.SemaphoreType.DMA,
            pltpu.SemaphoreType.DMA,
        ],
    )


def kernel(x, W0, adj_row, adj_col, adj_val):
    n, _ = x.shape
    d = W0.shape[1]
    e = adj_row.shape[0]

    ps_packed, edges = pl.pallas_call(
        _matmul_pack_body,
        out_shape=[
            jax.ShapeDtypeStruct((d // 2, n), jnp.int32),
            jax.ShapeDtypeStruct((3, e), jnp.int32),
        ],
    )(W0, x, adj_row, adj_col, adj_val)

    outT = _make_sc_edge_kernel(n, d, e)(ps_packed, edges)

    out = pl.pallas_call(
        _transpose_body,
        out_shape=jax.ShapeDtypeStruct((n, d), jnp.float32),
    )(outT)
    return out
